# Initial kernel scaffold; baseline (speedup 1.0000x reference)
import jax
import jax.numpy as jnp
from jax.experimental import pallas as pl


def kernel(out_x, a_x_x, edge_index, edge_attr, mask, f):
    raise NotImplementedError("write your pallas kernel here")


# speedup vs baseline: 28.8951x; 28.8951x over previous
import functools

import jax
import jax.numpy as jnp
from jax import lax
from jax.experimental import pallas as pl
from jax.experimental.pallas import tpu as pltpu
from jax.experimental.pallas import tpu_sc as plsc

NC = 2
NS = 16
NW = NC * NS
LANE = 16
CHUNK = 2048
CROWS = 16
CCOLS = CHUNK // CROWS


def _mesh():
    return plsc.VectorSubcoreMesh(core_axis_name="c", subcore_axis_name="s")


def _cparams():
    return pltpu.CompilerParams(needs_layout_passes=False)


def _worker_id():
    return lax.axis_index("c") * NS + lax.axis_index("s")


def _f32(shape):
    return jax.ShapeDtypeStruct(shape, jnp.float32)


def _zero_fill(ref, n):
    def body(i, _):
        ref[pl.ds(i * LANE, LANE)] = jnp.zeros((LANE,), jnp.float32)
        return 0
    lax.fori_loop(0, n // LANE, body, 0)


def _make_extract(npad):
    per = npad // NW
    groups = per // LANE

    def body(xf_hbm, af_hbm, xs_hbm, a0_hbm, inbuf, outbuf):
        w = _worker_id()
        iota = lax.iota(jnp.int32, LANE)
        for src_hbm, dst_hbm in ((xf_hbm, xs_hbm), (af_hbm, a0_hbm)):
            pltpu.sync_copy(src_hbm.at[pl.ds(w * per * 4, per * 4)], inbuf)

            def grp(g, _):
                idx = g * (4 * LANE) + 4 * iota
                outbuf[pl.ds(g * LANE, LANE)] = plsc.load_gather(inbuf, [idx])
                return 0
            lax.fori_loop(0, groups, grp, 0)
            pltpu.sync_copy(outbuf, dst_hbm.at[pl.ds(w * per, per)])

    return pl.kernel(
        body,
        out_type=(_f32((npad,)), _f32((npad,))),
        mesh=_mesh(), compiler_params=_cparams(),
        scratch_types=[
            pltpu.VMEM((per * 4,), jnp.float32),
            pltpu.VMEM((per,), jnp.float32),
        ],
    )


def _make_edge_pass(npad, n_edges, with_count):
    nchunk = n_edges // CHUNK
    ch_per_w = -(-nchunk // NW)
    sper = npad // NS

    def body(x_hbm, ei_hbm, ea_hbm, *rest):
        if with_count:
            (pa_hbm, pc_hbm, acc_sh, cnt_sh, x_local, src2d, dst2d,
             attr, vals, ones, sem, sem2) = rest
        else:
            (pa_hbm, acc_sh, x_local, src2d, dst2d,
             attr, vals, sem) = rest
        iobuf = attr.at[pl.ds(0, sper)]
        c = lax.axis_index("c")
        s = lax.axis_index("s")
        w = c * NS + s
        iota = lax.iota(jnp.int32, LANE)

        _zero_fill(attr, sper)
        pltpu.sync_copy(iobuf, acc_sh.at[pl.ds(s * sper, sper)])
        if with_count:
            pltpu.sync_copy(iobuf, cnt_sh.at[pl.ds(s * sper, sper)])

            def ob(i, _):
                r = i // (CCOLS // LANE)
                col = (i % (CCOLS // LANE)) * LANE
                ones[r, pl.ds(col, LANE)] = jnp.ones((LANE,), jnp.float32)
                return 0
            lax.fori_loop(0, CHUNK // LANE, ob, 0)

        pltpu.sync_copy(x_hbm, x_local)
        plsc.subcore_barrier()

        def chunk_body(j, _):
            chunk = w + NW * j

            @pl.when(chunk < nchunk)
            def _():
                pltpu.sync_copy(ei_hbm.at[0, chunk], src2d)
                pltpu.sync_copy(ei_hbm.at[1, chunk], dst2d)
                pltpu.sync_copy(
                    ea_hbm.at[pl.ds(chunk * (4 * CHUNK), 4 * CHUNK)], attr)

                def grp(g, _):
                    r = g // (CCOLS // LANE)
                    col = (g % (CCOLS // LANE)) * LANE
                    si = src2d[r, pl.ds(col, LANE)]
                    di = dst2d[r, pl.ds(col, LANE)]
                    sv = plsc.load_gather(x_local, [si])
                    dv = plsc.load_gather(x_local, [di])
                    abase = g * (4 * LANE) + 4 * iota
                    e0 = plsc.load_gather(attr, [abase])
                    if with_count:
                        wgt = 1.0 / e0
                    else:
                        e1 = plsc.load_gather(attr, [abase + 1])
                        wgt = (e0 + e1) / (e0 * e1)
                    vals[r, pl.ds(col, LANE)] = (dv - sv) * wgt
                    return 0
                lax.fori_loop(0, CHUNK // LANE, grp, 0)

                handles = []
                for r in range(CROWS):
                    handles.append(pltpu.async_copy(
                        vals.at[r], acc_sh.at[dst2d.at[r]], sem, add=True))
                    if with_count:
                        handles.append(pltpu.async_copy(
                            ones.at[r], cnt_sh.at[dst2d.at[r]], sem2,
                            add=True))
                for h in handles:
                    h.wait()
            return 0
        lax.fori_loop(0, ch_per_w, chunk_body, 0)

        plsc.subcore_barrier()
        pltpu.sync_copy(acc_sh.at[pl.ds(s * sper, sper)], iobuf)
        pltpu.sync_copy(iobuf, pa_hbm.at[pl.ds(c * npad + s * sper, sper)])
        if with_count:
            pltpu.sync_copy(cnt_sh.at[pl.ds(s * sper, sper)], iobuf)
            pltpu.sync_copy(iobuf, pc_hbm.at[pl.ds(c * npad + s * sper, sper)])

    out_type = (_f32((NC * npad,)), _f32((NC * npad,))) if with_count \
        else _f32((NC * npad,))
    scratch = [pltpu.VMEM_SHARED((npad,), jnp.float32)]
    if with_count:
        scratch.append(pltpu.VMEM_SHARED((npad,), jnp.float32))
    scratch += [
        pltpu.VMEM((npad,), jnp.float32),
        pltpu.VMEM((CROWS, CCOLS), jnp.int32),
        pltpu.VMEM((CROWS, CCOLS), jnp.int32),
        pltpu.VMEM((4 * CHUNK,), jnp.float32),
        pltpu.VMEM((CROWS, CCOLS), jnp.float32),
    ]
    if with_count:
        scratch.append(pltpu.VMEM((CROWS, CCOLS), jnp.float32))
    scratch.append(pltpu.SemaphoreType.DMA)
    if with_count:
        scratch.append(pltpu.SemaphoreType.DMA)

    return pl.kernel(
        body, out_type=out_type, mesh=_mesh(), compiler_params=_cparams(),
        scratch_types=scratch)


def _make_glue_t(npad):
    per = npad // NW
    groups = per // LANE

    def body(pa_hbm, pc_hbm, a0_hbm, t_hbm, cnt_hbm,
             b0, b1, c0, c1, ab, tb, cb):
        w = _worker_id()
        sl = pl.ds(w * per, per)
        sl1 = pl.ds(npad + w * per, per)
        pltpu.sync_copy(pa_hbm.at[sl], b0)
        pltpu.sync_copy(pa_hbm.at[sl1], b1)
        pltpu.sync_copy(pc_hbm.at[sl], c0)
        pltpu.sync_copy(pc_hbm.at[sl1], c1)
        pltpu.sync_copy(a0_hbm.at[sl], ab)

        def grp(g, _):
            d = pl.ds(g * LANE, LANE)
            cc = c0[d] + c1[d]
            t = ab[d] * (b0[d] + b1[d]) / jnp.maximum(cc, 1.0)
            tb[d] = t
            cb[d] = cc
            return 0
        lax.fori_loop(0, groups, grp, 0)
        pltpu.sync_copy(tb, t_hbm.at[sl])
        pltpu.sync_copy(cb, cnt_hbm.at[sl])

    return pl.kernel(
        body,
        out_type=(_f32((npad,)), _f32((npad,))),
        mesh=_mesh(), compiler_params=_cparams(),
        scratch_types=[pltpu.VMEM((per,), jnp.float32) for _ in range(7)],
    )


def _make_final(npad):
    per = npad // NW
    groups = per // LANE

    def body(pb_hbm, cnt_hbm, mask_hbm, fm_hbm, o_hbm,
             b0, b1, cb, mb, fb, ob):
        w = _worker_id()
        sl = pl.ds(w * per, per)
        pltpu.sync_copy(pb_hbm.at[sl], b0)
        pltpu.sync_copy(pb_hbm.at[pl.ds(npad + w * per, per)], b1)
        pltpu.sync_copy(cnt_hbm.at[sl], cb)
        pltpu.sync_copy(mask_hbm.at[sl], mb)
        pltpu.sync_copy(fm_hbm.at[sl], fb)

        def grp(g, _):
            d = pl.ds(g * LANE, LANE)
            v = (b0[d] + b1[d]) / jnp.maximum(cb[d], 1.0)
            ob[d] = v * mb[d] - fb[d]
            return 0
        lax.fori_loop(0, groups, grp, 0)
        pltpu.sync_copy(ob, o_hbm.at[sl])

    return pl.kernel(
        body,
        out_type=_f32((npad,)),
        mesh=_mesh(), compiler_params=_cparams(),
        scratch_types=[pltpu.VMEM((per,), jnp.float32) for _ in range(6)],
    )


def _pde_loss(out_x, a_x_x, edge_index, edge_attr, mask, f):
    n = out_x.shape[0]
    n_edges = edge_index.shape[1]
    npad = -(-n // (NW * LANE)) * (NW * LANE)

    pad_nodes = npad - n
    xf = jnp.pad(out_x.reshape(-1), (0, pad_nodes * 4))
    af = jnp.pad(a_x_x.reshape(-1), (0, pad_nodes * 4))
    mask_p = jnp.pad(mask, (0, pad_nodes))
    fm = mask_p * jnp.asarray(f, jnp.float32)
    ei4 = edge_index.reshape(2, n_edges // CHUNK, CROWS, CCOLS)
    ea_f = edge_attr.reshape(-1)

    xs, a0 = _make_extract(npad)(xf, af)
    pa, pc = _make_edge_pass(npad, n_edges, True)(xs, ei4, ea_f)
    t, cnt = _make_glue_t(npad)(pa, pc, a0)
    pb = _make_edge_pass(npad, n_edges, False)(t, ei4, ea_f)
    out = _make_final(npad)(pb, cnt, mask_p, fm)
    return out[:n]


def kernel(out_x, a_x_x, edge_index, edge_attr, mask, f):
    return _pde_loss(out_x, a_x_x, edge_index, edge_attr, mask, f)
